# baseline (speedup 1.0000x reference)
import jax
import jax.numpy as jnp
from jax import lax
from jax.experimental import pallas as pl
from jax.experimental.pallas import tpu as pltpu
from jax.experimental.pallas import tpu_sc as plsc

N_ROWS = 1000000
DIM = 32
BATCH = 16384

NC = 2
NS = 16
NW = NC * NS
RB = 31232
LTAIL = 512
TBASE = NW * RB + LTAIL
T64 = N_ROWS - TBASE
RMAX = RB + LTAIL
CPC = 512
NCOPY = RB // CPC
ISTG = 2048
NVREG = ISTG // 16
NICH = BATCH // ISTG
POSB = 14
PMASK = (1 << POSB) - 1
FIN_CAP = BATCH + 32
VTC = 128

_SENTINEL = 0x7FFFFFF0
_IMIN = -2147483648


def _sc_set_kernel(mem_hbm, idx_hbm, val_hbm, tmem_hbm, out_hbm, tout_hbm,
                   istg_v, tab_v, fin_v, cbuf0_v, cbuf1_v, cbuf2_v, cbuf3_v,
                   vt0_v, vt1_v, tbuf_v,
                   cg0_sem, cg1_sem, cg2_sem, cg3_sem,
                   cs0_sem, cs1_sem, cs2_sem, cs3_sem,
                   vs0_sem, vs1_sem):
    w = lax.axis_index("s") * NC + lax.axis_index("c")
    lo = pl.multiple_of(w * RB, 128)
    nrows = jnp.where(w == NW - 1, RB + LTAIL, RB)
    iota = lax.iota(jnp.int32, 16)
    cbufs = (cbuf0_v, cbuf1_v, cbuf2_v, cbuf3_v)
    cgsems = (cg0_sem, cg1_sem, cg2_sem, cg3_sem)
    csems = (cs0_sem, cs1_sem, cs2_sem, cs3_sem)
    vts = (vt0_v, vt1_v)
    vsems = (vs0_sem, vs1_sem)

    for j in range(2):
        pltpu.async_copy(
            mem_hbm.at[:, pl.ds(pl.multiple_of(lo + j * CPC, 128), CPC)],
            cbufs[j], cgsems[j])

    neg1 = jnp.full((16,), -1, jnp.int32)

    def memset(i, carry):
        tab_v[pl.ds(i * 16, 16)] = neg1
        return carry

    lax.fori_loop(0, RMAX // 16, memset, jnp.int32(0))

    def make_pass_a(ci):
        def pass_a(i, carry):
            base = i * 16
            vec = istg_v[pl.ds(base, 16)]
            loc = vec - lo
            tail_hit = (vec >= TBASE) & (w == 0)
            loc = jnp.where(tail_hit, RB + (vec - TBASE), loc)
            valid = ((loc >= 0) & (loc < nrows)) | tail_hit
            key = jnp.where(valid, (loc << 4) | iota, _SENTINEL)
            pos = jnp.where(valid, ci * ISTG + base + iota, -1)
            sk, sv = plsc.sort_key_val(key, pos)
            nbr = jnp.minimum(iota + 1, 15)
            knext = sk.at[nbr].get(mode="promise_in_bounds")
            run_last = ((sk >> 4) != (knext >> 4)) | (iota == 15)
            m = run_last & (sv >= 0)
            plsc.store_scatter(tab_v, [sk >> 4], sv, mask=m)
            return carry
        return pass_a

    with jax.named_scope("pass_a"):
        for ci in range(NICH):
            pltpu.sync_copy(idx_hbm.at[pl.ds(ci * ISTG, ISTG)], istg_v)
            lax.fori_loop(0, NVREG, make_pass_a(ci), jnp.int32(0))

    def scan(i, cnt):
        v = tab_v[pl.ds(i * 16, 16)]
        m = v >= 0
        packed = ((i * 16 + iota) << POSB) | (v & PMASK)
        plsc.store_compressed(fin_v.at[pl.ds(cnt, 16)], packed, mask=m)
        return cnt + jnp.max(plsc.all_reduce_population_count(m))

    with jax.named_scope("tab_scan"):
        cnt = lax.fori_loop(0, RMAX // 16, scan, jnp.int32(0))

    def read_packed(p):
        base = (p >> 4) << 4
        va = fin_v[pl.ds(base, 16)]
        x = jnp.max(jnp.where(iota == (p - base), va, _IMIN))
        return jnp.where(p < cnt, x, _SENTINEL)

    def issue_vtile(packed, slot):
        vb = pl.multiple_of(((packed & PMASK) >> 7) * VTC, 128)
        pltpu.async_copy(val_hbm.at[:, pl.ds(vb, VTC)], vts[slot],
                         vsems[slot])

    def issue_next(nxt, wp1):
        @pl.when(nxt != _SENTINEL)
        def _():
            @pl.when(wp1 % 2 == 0)
            def _():
                issue_vtile(nxt, 0)

            @pl.when(wp1 % 2 == 1)
            def _():
                issue_vtile(nxt, 1)

    def merge_cur(cur, wptr, base_loc, cb):
        tgt = jnp.zeros((16,), jnp.int32) + ((cur >> POSB) - base_loc)
        vt_col = jnp.zeros((16,), jnp.int32) + ((cur & PMASK) & (VTC - 1))
        for slot in range(2):
            @pl.when(wptr % 2 == slot)
            def _():
                pltpu.make_async_copy(val_hbm.at[:, pl.ds(0, VTC)], vts[slot],
                                      vsems[slot]).wait()
                lo16 = plsc.load_gather(vts[slot], [iota, vt_col])
                hi16 = plsc.load_gather(vts[slot], [iota + 16, vt_col])
                plsc.store_scatter(cb, [iota, tgt], lo16)
                plsc.store_scatter(cb, [iota + 16, tgt], hi16)

    def merge_chunk(state, base_loc, hi_loc, cb):
        def cond(st):
            _, cur = st
            return (cur >> POSB) < hi_loc

        def body(st):
            wptr, cur = st
            nxt = read_packed(wptr + 1)
            issue_next(nxt, wptr + 1)
            merge_cur(cur, wptr, base_loc, cb)
            return (wptr + 1, nxt)

        return lax.while_loop(cond, body, state)

    cur0 = read_packed(jnp.int32(0))

    @pl.when(cur0 != _SENTINEL)
    def _():
        issue_vtile(cur0, 0)

    def do_chunk(k, j, state, ncopy):
        base = pl.multiple_of(lo + k * CPC, 128)
        base_loc = k * CPC

        pltpu.make_async_copy(mem_hbm.at[:, pl.ds(base, CPC)], cbufs[j],
                              cgsems[j]).wait()
        state = merge_chunk(state, base_loc, base_loc + CPC, cbufs[j])
        pltpu.async_copy(cbufs[j], out_hbm.at[:, pl.ds(base, CPC)], csems[j])

        b2 = (j + 2) % 4

        @pl.when(k >= 2)
        def _():
            pltpu.make_async_copy(cbufs[b2], out_hbm.at[:, pl.ds(base, CPC)],
                                  csems[b2]).wait()

        @pl.when(k + 2 < ncopy)
        def _():
            base2 = pl.multiple_of(lo + (k + 2) * CPC, 128)
            pltpu.async_copy(mem_hbm.at[:, pl.ds(base2, CPC)], cbufs[b2],
                             cgsems[b2])
        return state

    nc_w = jnp.where(w == NW - 1, NCOPY + 1, NCOPY)

    with jax.named_scope("copy_merge"):
        def trip(t, state):
            for j in range(4):
                state = do_chunk(t * 4 + j, j, state, nc_w)
            return state

        state = lax.fori_loop(0, NCOPY // 4, trip, (jnp.int32(0), cur0))
        state = do_chunk(jnp.int32(NCOPY - 1), (NCOPY - 1) % 4, state, nc_w)

        @pl.when(w == NW - 1)
        def _extra():
            do_chunk(jnp.int32(NCOPY), NCOPY % 4, state, nc_w)

        def _drain_sem(i):
            pltpu.make_async_copy(cbufs[i % 4], out_hbm.at[:, pl.ds(lo, CPC)],
                                  csems[i % 4]).wait()

        @pl.when(w != NW - 1)
        def _drain():
            _drain_sem(NCOPY - 2)
            _drain_sem(NCOPY - 1)

        @pl.when(w == NW - 1)
        def _drain_extra():
            _drain_sem(NCOPY - 1)
            _drain_sem(NCOPY)

        @pl.when(w == 0)
        def _tail():
            pltpu.async_copy(tmem_hbm, tbuf_v, cg1_sem).wait()
            merge_chunk(state, RB, RB + T64, tbuf_v)
            pltpu.async_copy(tbuf_v, tout_hbm, cs1_sem).wait()


@jax.jit
def _sc_set(memory, node_idxs, values):
    out_t, out_tail = pl.kernel(
        _sc_set_kernel,
        out_type=(jax.ShapeDtypeStruct((DIM, N_ROWS), jnp.float32),
                  jax.ShapeDtypeStruct((DIM, T64), jnp.float32)),
        mesh=plsc.VectorSubcoreMesh(core_axis_name="c", subcore_axis_name="s"),
        compiler_params=pltpu.CompilerParams(
            needs_layout_passes=False, use_tc_tiling_on_sc=True),
        scratch_types=[
            pltpu.VMEM((ISTG,), jnp.int32),
            pltpu.VMEM((RMAX,), jnp.int32),
            pltpu.VMEM((FIN_CAP,), jnp.int32),
            pltpu.VMEM((DIM, CPC), jnp.float32),
            pltpu.VMEM((DIM, CPC), jnp.float32),
            pltpu.VMEM((DIM, CPC), jnp.float32),
            pltpu.VMEM((DIM, CPC), jnp.float32),
            pltpu.VMEM((DIM, VTC), jnp.float32),
            pltpu.VMEM((DIM, VTC), jnp.float32),
            pltpu.VMEM((DIM, T64), jnp.float32),
            pltpu.SemaphoreType.DMA,
            pltpu.SemaphoreType.DMA,
            pltpu.SemaphoreType.DMA,
            pltpu.SemaphoreType.DMA,
            pltpu.SemaphoreType.DMA,
            pltpu.SemaphoreType.DMA,
            pltpu.SemaphoreType.DMA,
            pltpu.SemaphoreType.DMA,
            pltpu.SemaphoreType.DMA,
            pltpu.SemaphoreType.DMA,
        ],
    )(memory.T, node_idxs, values.T,
      lax.slice(memory, (TBASE, 0), (N_ROWS, DIM)).T)
    return lax.dynamic_update_slice(out_t.T, out_tail.T, (TBASE, 0))


def kernel(memory, node_idxs, values):
    return _sc_set(memory, node_idxs, values)
